# Initial kernel scaffold; baseline (speedup 1.0000x reference)
import jax
import jax.numpy as jnp
from jax.experimental import pallas as pl


def kernel(x, weight):
    raise NotImplementedError("write your pallas kernel here")


# speedup vs baseline: 3.6090x; 3.6090x over previous
import functools

import jax
import jax.numpy as jnp
from jax import lax
from jax.experimental import pallas as pl
from jax.experimental.pallas import tpu as pltpu
from jax.experimental.pallas import tpu_sc as plsc

_VOCAB = 56
_D = 64
_ROWS, _COLS = 16384, 200
_B = _ROWS * _COLS
_NPAIR = _B // 2
_PT_ROWS = _VOCAB * _VOCAB
_NC, _NS = 2, 16
_NW = _NC * _NS
_PT_PER_S = _PT_ROWS // _NS
_SUB = 128
_NSUB = 2
_CHUNK = _SUB * _NSUB
_PAIRS_PER_W = _NPAIR // _NW
_STEPS = _PAIRS_PER_W // _CHUNK
_IDXR_PER_W = _PAIRS_PER_W // _SUB

_mesh = plsc.VectorSubcoreMesh(
    core_axis_name="c", subcore_axis_name="s",
    num_cores=_NC, num_subcores=_NS,
)


def _pairidx_body(even_ref, odd_ref, out_ref):
    out_ref[...] = even_ref[...] * _VOCAB + odd_ref[...]


_pairidx = pl.pallas_call(
    _pairidx_body,
    out_shape=jax.ShapeDtypeStruct((_NPAIR // _SUB, _SUB), jnp.int32),
    grid=(8,),
    in_specs=[
        pl.BlockSpec((_NPAIR // _SUB // 8, _SUB), lambda i: (i, 0)),
        pl.BlockSpec((_NPAIR // _SUB // 8, _SUB), lambda i: (i, 0)),
    ],
    out_specs=pl.BlockSpec((_NPAIR // _SUB // 8, _SUB), lambda i: (i, 0)),
)


@functools.partial(
    pl.kernel,
    out_type=jax.ShapeDtypeStruct((_NPAIR, 2 * _D), jnp.float32),
    mesh=_mesh,
    scratch_types=[
        pltpu.VMEM((_VOCAB * _D,), jnp.float32),
        pltpu.VMEM((_PT_PER_S, 2 * _D), jnp.float32),
        pltpu.VMEM((_NSUB, _SUB), jnp.int32),
        pltpu.VMEM((_CHUNK, 2 * _D), jnp.float32),
        pltpu.VMEM_SHARED((_PT_ROWS, 2 * _D), jnp.float32),
        pltpu.SemaphoreType.DMA,
    ],
)
def _embed(pidx_hbm, wflat_hbm, out_hbm,
           wv, ptbuild, pidx_v, rows_v, pt_sh, sem):
    cid = lax.axis_index("c")
    sid = lax.axis_index("s")
    wid = sid * _NC + cid

    pltpu.sync_copy(wflat_hbm, wv)
    p0 = sid * _PT_PER_S

    def build_row(r, carry):
        a = (p0 + r) // _VOCAB
        b = (p0 + r) % _VOCAB
        for q in range(4):
            ptbuild[r, pl.ds(q * 16, 16)] = wv[pl.ds(a * _D + q * 16, 16)]
            ptbuild[r, pl.ds(_D + q * 16, 16)] = wv[pl.ds(b * _D + q * 16, 16)]
        return carry

    lax.fori_loop(0, _PT_PER_S, build_row, 0)
    pltpu.sync_copy(ptbuild, pt_sh.at[pl.ds(p0, _PT_PER_S)])
    plsc.subcore_barrier()

    idxr0 = wid * _IDXR_PER_W

    def body(g, carry):
        r0 = idxr0 + g * _NSUB
        pltpu.sync_copy(pidx_hbm.at[pl.ds(r0, _NSUB)], pidx_v)
        descs = [
            pltpu.async_copy(
                pt_sh.at[pidx_v.at[j]],
                rows_v.at[pl.ds(j * _SUB, _SUB)],
                sem,
            )
            for j in range(_NSUB)
        ]
        for d in descs:
            d.wait()
        pltpu.sync_copy(rows_v, out_hbm.at[pl.ds(r0 * _SUB, _CHUNK)])
        return carry

    lax.fori_loop(0, _STEPS, body, 0)


def kernel(x, weight):
    x3 = x.reshape(_NPAIR, 2).astype(jnp.int32)
    even = x3[:, 0].reshape(_NPAIR // _SUB, _SUB)
    odd = x3[:, 1].reshape(_NPAIR // _SUB, _SUB)
    pidx = _pairidx(even, odd)
    wflat = weight.reshape(_VOCAB * _D)
    out = _embed(pidx, wflat)
    return out.reshape(_ROWS, _COLS, _D)


# speedup vs baseline: 3.8680x; 1.0717x over previous
import functools

import jax
import jax.numpy as jnp
from jax import lax
from jax.experimental import pallas as pl
from jax.experimental.pallas import tpu as pltpu
from jax.experimental.pallas import tpu_sc as plsc

_VOCAB = 56
_D = 64
_ROWS, _COLS = 16384, 200
_B = _ROWS * _COLS
_NPAIR = _B // 2
_PT_ROWS = _VOCAB * _VOCAB
_NC, _NS = 2, 16
_NW = _NC * _NS
_PT_PER_S = _PT_ROWS // _NS
_SUB = 128
_NSUB = 2
_CHUNK = _SUB * _NSUB
_PAIRS_PER_W = _NPAIR // _NW
_STEPS = _PAIRS_PER_W // _CHUNK
_IDXR_PER_W = _PAIRS_PER_W // _SUB

_mesh = plsc.VectorSubcoreMesh(
    core_axis_name="c", subcore_axis_name="s",
    num_cores=_NC, num_subcores=_NS,
)


def _pairidx_body(even_ref, odd_ref, out_ref):
    out_ref[...] = even_ref[...] * _VOCAB + odd_ref[...]


_pairidx = pl.pallas_call(
    _pairidx_body,
    out_shape=jax.ShapeDtypeStruct((_NPAIR // _SUB, _SUB), jnp.int32),
    grid=(8,),
    in_specs=[
        pl.BlockSpec((_NPAIR // _SUB // 8, _SUB), lambda i: (i, 0)),
        pl.BlockSpec((_NPAIR // _SUB // 8, _SUB), lambda i: (i, 0)),
    ],
    out_specs=pl.BlockSpec((_NPAIR // _SUB // 8, _SUB), lambda i: (i, 0)),
)


@functools.partial(
    pl.kernel,
    out_type=jax.ShapeDtypeStruct((_NPAIR, 2 * _D), jnp.float32),
    mesh=_mesh,
    scratch_types=[
        pltpu.VMEM((_VOCAB * _D,), jnp.float32),
        pltpu.VMEM((_PT_PER_S, 2 * _D), jnp.float32),
        pltpu.VMEM((2, _NSUB, _SUB), jnp.int32),
        pltpu.VMEM((2, _CHUNK, 2 * _D), jnp.float32),
        pltpu.VMEM_SHARED((_PT_ROWS, 2 * _D), jnp.float32),
        pltpu.SemaphoreType.DMA,
        pltpu.SemaphoreType.DMA,
        pltpu.SemaphoreType.DMA,
    ],
)
def _embed(pidx_hbm, wflat_hbm, out_hbm,
           wv, ptbuild, pidx_v, rows_v, pt_sh, gsem, osem0, osem1):
    cid = lax.axis_index("c")
    sid = lax.axis_index("s")
    wid = sid * _NC + cid

    pltpu.sync_copy(wflat_hbm, wv)
    p0 = sid * _PT_PER_S

    def build_row(r, carry):
        a = (p0 + r) // _VOCAB
        b = (p0 + r) % _VOCAB
        for q in range(4):
            ptbuild[r, pl.ds(q * 16, 16)] = wv[pl.ds(a * _D + q * 16, 16)]
            ptbuild[r, pl.ds(_D + q * 16, 16)] = wv[pl.ds(b * _D + q * 16, 16)]
        return carry

    lax.fori_loop(0, _PT_PER_S, build_row, 0)
    pltpu.sync_copy(ptbuild, pt_sh.at[pl.ds(p0, _PT_PER_S)])
    plsc.subcore_barrier()

    idxr0 = wid * _IDXR_PER_W
    osems = (osem0, osem1)

    def body(h, carry):
        for b in range(2):
            g = 2 * h + b
            r0 = idxr0 + g * _NSUB
            buf = rows_v.at[b]

            @pl.when(h >= 1)
            def _():
                pltpu.make_async_copy(
                    buf, out_hbm.at[pl.ds(0, _CHUNK)], osems[b]
                ).wait()

            pltpu.sync_copy(pidx_hbm.at[pl.ds(r0, _NSUB)], pidx_v.at[b])
            descs = [
                pltpu.async_copy(
                    pt_sh.at[pidx_v.at[b].at[j]],
                    buf.at[pl.ds(j * _SUB, _SUB)],
                    gsem,
                )
                for j in range(_NSUB)
            ]
            for d in descs:
                d.wait()
            pltpu.async_copy(buf, out_hbm.at[pl.ds(r0 * _SUB, _CHUNK)], osems[b])
        return carry

    lax.fori_loop(0, _STEPS // 2, body, 0)
    for b in range(2):
        pltpu.make_async_copy(
            rows_v.at[b], out_hbm.at[pl.ds(0, _CHUNK)], osems[b]
        ).wait()


def kernel(x, weight):
    x3 = x.reshape(_NPAIR, 2).astype(jnp.int32)
    even = x3[:, 0].reshape(_NPAIR // _SUB, _SUB)
    odd = x3[:, 1].reshape(_NPAIR // _SUB, _SUB)
    pidx = _pairidx(even, odd)
    wflat = weight.reshape(_VOCAB * _D)
    out = _embed(pidx, wflat)
    return out.reshape(_ROWS, _COLS, _D)


# speedup vs baseline: 3.9622x; 1.0244x over previous
import functools

import jax
import jax.numpy as jnp
from jax import lax
from jax.experimental import pallas as pl
from jax.experimental.pallas import tpu as pltpu
from jax.experimental.pallas import tpu_sc as plsc

_VOCAB = 56
_D = 64
_ROWS, _COLS = 16384, 200
_B = _ROWS * _COLS
_NPAIR = _B // 2
_PT_ROWS = _VOCAB * _VOCAB
_NC, _NS = 2, 16
_NW = _NC * _NS
_PT_PER_S = _PT_ROWS // _NS
_SUB = 128
_NBUF = 4
_PAIRS_PER_W = _NPAIR // _NW
_IDXR_PER_W = _PAIRS_PER_W // _SUB

_mesh = plsc.VectorSubcoreMesh(
    core_axis_name="c", subcore_axis_name="s",
    num_cores=_NC, num_subcores=_NS,
)


def _pairidx_body(even_ref, odd_ref, out_ref):
    out_ref[...] = even_ref[...] * _VOCAB + odd_ref[...]


_pairidx = pl.pallas_call(
    _pairidx_body,
    out_shape=jax.ShapeDtypeStruct((_NPAIR // _SUB, _SUB), jnp.int32),
    grid=(8,),
    in_specs=[
        pl.BlockSpec((_NPAIR // _SUB // 8, _SUB), lambda i: (i, 0)),
        pl.BlockSpec((_NPAIR // _SUB // 8, _SUB), lambda i: (i, 0)),
    ],
    out_specs=pl.BlockSpec((_NPAIR // _SUB // 8, _SUB), lambda i: (i, 0)),
)


@functools.partial(
    pl.kernel,
    out_type=jax.ShapeDtypeStruct((_NPAIR, 2 * _D), jnp.float32),
    mesh=_mesh,
    scratch_types=[
        pltpu.VMEM((_VOCAB * _D,), jnp.float32),
        pltpu.VMEM((_IDXR_PER_W // 2, _SUB), jnp.int32),
        pltpu.VMEM((_NBUF * _SUB, 2 * _D), jnp.float32),
        pltpu.VMEM_SHARED((_PT_ROWS, 2 * _D), jnp.float32),
        pltpu.SemaphoreType.DMA,
        pltpu.SemaphoreType.DMA,
        pltpu.SemaphoreType.DMA,
        pltpu.SemaphoreType.DMA,
        pltpu.SemaphoreType.DMA,
        pltpu.SemaphoreType.DMA,
    ],
)
def _embed(pidx_hbm, wflat_hbm, out_hbm,
           wv, idxall, rows_v, pt_sh, isem, gsem, osem0, osem1, osem2, osem3):
    cid = lax.axis_index("c")
    sid = lax.axis_index("s")
    wid = sid * _NC + cid
    idxr0 = wid * _IDXR_PER_W
    pair0 = wid * _PAIRS_PER_W
    osems = (osem0, osem1, osem2, osem3)

    _HALF = _IDXR_PER_W // 2
    idesc = pltpu.async_copy(
        pidx_hbm.at[pl.ds(idxr0, _HALF)], idxall, isem)

    pltpu.sync_copy(wflat_hbm, wv)
    p0 = sid * _PT_PER_S

    def build_row(r, carry):
        a = (p0 + r) // _VOCAB
        b = (p0 + r) % _VOCAB
        for q in range(4):
            rows_v[r, pl.ds(q * 16, 16)] = wv[pl.ds(a * _D + q * 16, 16)]
            rows_v[r, pl.ds(_D + q * 16, 16)] = wv[pl.ds(b * _D + q * 16, 16)]
        return carry

    lax.fori_loop(0, _PT_PER_S, build_row, 0)
    pltpu.sync_copy(rows_v.at[pl.ds(0, _PT_PER_S)], pt_sh.at[pl.ds(p0, _PT_PER_S)])
    idesc.wait()
    plsc.subcore_barrier()

    def fire_gather(r, b):
        pltpu.async_copy(
            pt_sh.at[idxall.at[r]],
            rows_v.at[pl.ds(b * _SUB, _SUB)],
            gsem,
        )

    def fire_write(g, b):
        pltpu.async_copy(
            rows_v.at[pl.ds(b * _SUB, _SUB)],
            out_hbm.at[pl.ds(pair0 + g * _SUB, _SUB)],
            osems[b],
        )

    def drain_write(b):
        pltpu.make_async_copy(
            rows_v.at[pl.ds(b * _SUB, _SUB)],
            out_hbm.at[pl.ds(0, _SUB)],
            osems[b],
        ).wait()

    def drain_gather(r, b):
        pltpu.make_async_copy(
            pt_sh.at[idxall.at[r]],
            rows_v.at[pl.ds(b * _SUB, _SUB)],
            gsem,
        ).wait()

    for block in range(2):
        g0 = block * _HALF
        if block > 0:
            pltpu.sync_copy(
                pidx_hbm.at[pl.ds(idxr0 + g0, _HALF)], idxall)

        def body(h, carry, g0=g0):
            for b in range(_NBUF):
                r = _NBUF * h + b

                @pl.when(h >= 1)
                def _():
                    drain_write(b)

                fire_gather(r, b)
                pb = (b - 1) % _NBUF
                if b == 0:
                    @pl.when(h >= 1)
                    def _():
                        drain_gather(r - 1, pb)
                        fire_write(g0 + r - 1, pb)
                else:
                    drain_gather(r - 1, pb)
                    fire_write(g0 + r - 1, pb)
            return carry

        lax.fori_loop(0, _HALF // _NBUF, body, 0)
        last_b = _NBUF - 1
        drain_gather(_HALF - 1, last_b)
        fire_write(g0 + _HALF - 1, last_b)
        for b in range(_NBUF):
            drain_write(b)


def kernel(x, weight):
    x3 = x.reshape(_NPAIR, 2).astype(jnp.int32)
    even = x3[:, 0].reshape(_NPAIR // _SUB, _SUB)
    odd = x3[:, 1].reshape(_NPAIR // _SUB, _SUB)
    pidx = _pairidx(even, odd)
    wflat = weight.reshape(_VOCAB * _D)
    out = _embed(pidx, wflat)
    return out.reshape(_ROWS, _COLS, _D)


# speedup vs baseline: 5.8209x; 1.4691x over previous
import functools

import jax
import jax.numpy as jnp
from jax import lax
from jax.experimental import pallas as pl
from jax.experimental.pallas import tpu as pltpu
from jax.experimental.pallas import tpu_sc as plsc

_VOCAB = 56
_D = 64
_ROWS, _COLS = 16384, 200
_NC, _NS = 2, 16
_NW = _NC * _NS
_NBUF = 4
_ROWS_PER_W = _ROWS // _NW
_IB = 64
_NBLK = _ROWS_PER_W // _IB
_G0, _G1 = 128, _COLS - 128

_mesh = plsc.VectorSubcoreMesh(
    core_axis_name="c", subcore_axis_name="s",
    num_cores=_NC, num_subcores=_NS,
)


@functools.partial(
    pl.kernel,
    out_type=jax.ShapeDtypeStruct((_ROWS, _COLS, _D), jnp.float32),
    mesh=_mesh,
    compiler_params=pltpu.CompilerParams(use_tc_tiling_on_sc=False),
    scratch_types=[
        pltpu.VMEM((_VOCAB * _D,), jnp.float32),
        pltpu.VMEM((_IB, _COLS), jnp.int32),
        pltpu.VMEM((_NBUF * _COLS, _D), jnp.float32),
        pltpu.VMEM_SHARED((_VOCAB, _D), jnp.float32),
        pltpu.SemaphoreType.DMA,
        pltpu.SemaphoreType.DMA,
        pltpu.SemaphoreType.DMA,
        pltpu.SemaphoreType.DMA,
        pltpu.SemaphoreType.DMA,
    ],
)
def _embed(x_hbm, wflat_hbm, out_hbm,
           wv, idxv, rows_v, tab_sh, gsem, osem0, osem1, osem2, osem3):
    cid = lax.axis_index("c")
    sid = lax.axis_index("s")
    wid = sid * _NC + cid
    row0 = wid * _ROWS_PER_W
    osems = (osem0, osem1, osem2, osem3)

    @pl.when(sid == 0)
    def _():
        pltpu.sync_copy(wflat_hbm, wv)

        def build_row(v, carry):
            for q in range(4):
                rows_v[v, pl.ds(q * 16, 16)] = wv[pl.ds(v * _D + q * 16, 16)]
            return carry

        lax.fori_loop(0, _VOCAB, build_row, 0)
        pltpu.sync_copy(rows_v.at[pl.ds(0, _VOCAB)], tab_sh)

    plsc.subcore_barrier()

    def fire_gather(r, b):
        pltpu.async_copy(
            tab_sh.at[idxv.at[r].at[pl.ds(0, _G0)]],
            rows_v.at[pl.ds(b * _COLS, _G0)],
            gsem,
        )
        pltpu.async_copy(
            tab_sh.at[idxv.at[r].at[pl.ds(_G0, _G1)]],
            rows_v.at[pl.ds(b * _COLS + _G0, _G1)],
            gsem,
        )

    def drain_gather(r, b):
        pltpu.make_async_copy(
            tab_sh.at[idxv.at[r].at[pl.ds(0, _G0)]],
            rows_v.at[pl.ds(b * _COLS, _G0)],
            gsem,
        ).wait()
        pltpu.make_async_copy(
            tab_sh.at[idxv.at[r].at[pl.ds(_G0, _G1)]],
            rows_v.at[pl.ds(b * _COLS + _G0, _G1)],
            gsem,
        ).wait()

    def fire_write(i0, b):
        pltpu.async_copy(
            rows_v.at[pl.ds(b * _COLS, _COLS)],
            out_hbm.at[i0],
            osems[b],
        )

    def drain_write(b):
        pltpu.make_async_copy(
            rows_v.at[pl.ds(b * _COLS, _COLS)],
            out_hbm.at[0],
            osems[b],
        ).wait()

    def block(j, carry):
        pltpu.sync_copy(x_hbm.at[pl.ds(row0 + j * _IB, _IB)], idxv)

        def body(h, c2):
            for b in range(_NBUF):
                r = _NBUF * h + b

                @pl.when(jnp.logical_or(j > 0, h >= 1))
                def _():
                    drain_write(b)

                fire_gather(r, b)
                pb = (b - 1) % _NBUF
                if b == 0:
                    @pl.when(h >= 1)
                    def _():
                        drain_gather(r - 1, pb)
                        fire_write(row0 + j * _IB + r - 1, pb)
                else:
                    drain_gather(r - 1, pb)
                    fire_write(row0 + j * _IB + r - 1, pb)
            return c2

        lax.fori_loop(0, _IB // _NBUF, body, 0)
        drain_gather(_IB - 1, _NBUF - 1)
        fire_write(row0 + j * _IB + _IB - 1, _NBUF - 1)
        return carry

    lax.fori_loop(0, _NBLK, block, 0)
    for b in range(_NBUF):
        drain_write(b)


def kernel(x, weight):
    return _embed(x.astype(jnp.int32), weight.reshape(_VOCAB * _D))
